# Initial kernel scaffold; baseline (speedup 1.0000x reference)
import jax
import jax.numpy as jnp
from jax.experimental import pallas as pl


def kernel(x, edge_index, W_pre, b_pre, W_post, b_post):
    raise NotImplementedError("write your pallas kernel here")


# speedup vs baseline: 7.5701x; 7.5701x over previous
import functools

import jax
import jax.numpy as jnp
from jax import lax
from jax.experimental import pallas as pl
from jax.experimental.pallas import tpu as pltpu
from jax.experimental.pallas import tpu_sc as plsc

N = 10000
E = 320000
D = 128
DH = 64
STEPS = 4

NC = 2
NS = 16
NW = NC * NS
EPT = E // NW
K = 80
CH = EPT // K
NP = 10240
RPT = NP // NS
SR = 128
NB = NP // SR
BPT = RPT // SR
DW = 16

_mesh = plsc.VectorSubcoreMesh(core_axis_name="c", subcore_axis_name="s")


@functools.partial(
    pl.kernel,
    out_type=jax.ShapeDtypeStruct((NC, NP, DW), jnp.float32),
    mesh=_mesh,
    compiler_params=pltpu.CompilerParams(use_tc_tiling_on_sc=False),
    scratch_types=[
        pltpu.VMEM((CH, K), jnp.int32),
        pltpu.VMEM((K, DW), jnp.float32),
        pltpu.VMEM((SR, DW), jnp.float32),
        pltpu.VMEM_SHARED((NP, DW), jnp.float32),
    ],
)
def _deg_sc(dst_hbm, ones_hbm, zeros_hbm, out_hbm, dst_v, ones_v, stage_v, acc):
    cid = lax.axis_index("c")
    sid = lax.axis_index("s")
    wid = cid * NS + sid
    pltpu.sync_copy(dst_hbm.at[wid], dst_v)
    pltpu.sync_copy(ones_hbm, ones_v)
    for j in range(BPT):
        b = sid * BPT + j
        r0 = pl.multiple_of(b * SR, SR)
        pltpu.sync_copy(zeros_hbm.at[b], stage_v)
        pltpu.sync_copy(stage_v, acc.at[pl.ds(r0, SR)])
    plsc.subcore_barrier()

    def body(j, carry):
        pltpu.sync_copy(ones_v, acc.at[dst_v.at[j]], add=True)
        return carry

    lax.fori_loop(0, CH, body, 0)
    plsc.subcore_barrier()
    for j in range(BPT):
        b = sid * BPT + j
        r0 = pl.multiple_of(b * SR, SR)
        pltpu.sync_copy(acc.at[pl.ds(r0, SR)], stage_v)
        pltpu.sync_copy(stage_v, out_hbm.at[cid, pl.ds(r0, SR)])


@functools.partial(
    pl.kernel,
    out_type=(jax.ShapeDtypeStruct((NC, NP, DH), jnp.float32),
              jax.ShapeDtypeStruct((NC, NP, DH), jnp.float32)),
    mesh=_mesh,
    compiler_params=pltpu.CompilerParams(use_tc_tiling_on_sc=False),
    scratch_types=[
        pltpu.VMEM((CH, K), jnp.int32),
        pltpu.VMEM((CH, K), jnp.int32),
        pltpu.VMEM((K, DH), jnp.float32),
        pltpu.VMEM((SR, DH), jnp.float32),
        pltpu.VMEM_SHARED((NP, DH), jnp.float32),
        pltpu.SemaphoreType.DMA,
    ],
)
def _prop_sc(va_hbm, vb_hbm, src_hbm, dst_hbm, zeros_hbm, outa_hbm, outb_hbm,
             src_v, dst_v, rows_v, stage_v, acc, sem):
    cid = lax.axis_index("c")
    sid = lax.axis_index("s")
    wid = cid * NS + sid
    pltpu.sync_copy(src_hbm.at[wid], src_v)
    pltpu.sync_copy(dst_hbm.at[wid], dst_v)
    for v_hbm, out_hbm in ((va_hbm, outa_hbm), (vb_hbm, outb_hbm)):
        for j in range(BPT):
            b = sid * BPT + j
            r0 = pl.multiple_of(b * SR, SR)
            pltpu.sync_copy(zeros_hbm.at[b], stage_v)
            pltpu.sync_copy(stage_v, acc.at[pl.ds(r0, SR)])
        plsc.subcore_barrier()

        def body(j, carry):
            pltpu.async_copy(v_hbm.at[src_v.at[j]], rows_v, sem).wait()
            pltpu.sync_copy(rows_v, acc.at[dst_v.at[j]], add=True)
            return carry

        lax.fori_loop(0, CH, body, 0)
        plsc.subcore_barrier()
        for j in range(BPT):
            b = sid * BPT + j
            r0 = pl.multiple_of(b * SR, SR)
            pltpu.sync_copy(acc.at[pl.ds(r0, SR)], stage_v)
            pltpu.sync_copy(stage_v, out_hbm.at[cid, pl.ds(r0, SR)])


def _deg_col(deg16):
    d = (jnp.max(deg16[0, :N], axis=1, keepdims=True)
         + jnp.max(deg16[1, :N], axis=1, keepdims=True))
    return jnp.maximum(d, 1.0)


def _prep_body(x_ref, w_ref, b_ref, deg_ref, hidden_ref, v0a_ref, v0b_ref):
    h = jnp.dot(x_ref[...], w_ref[...], preferred_element_type=jnp.float32)
    h = h + b_ref[...][None, :]
    norm = lax.rsqrt(_deg_col(deg_ref[...]))
    hidden_ref[...] = h
    v0 = h * norm
    v0a_ref[...] = v0[:, :DH]
    v0b_ref[...] = v0[:, DH:]


def _comb_body(sa_ref, sb_ref, deg_ref, v0a_ref, v0b_ref, va_ref, vb_ref):
    inv = 0.5 / _deg_col(deg_ref[...])
    va_ref[...] = inv * (sa_ref[0, :N] + sa_ref[1, :N]) + 0.5 * v0a_ref[...]
    vb_ref[...] = inv * (sb_ref[0, :N] + sb_ref[1, :N]) + 0.5 * v0b_ref[...]


def _final_body(sa_ref, sb_ref, deg_ref, hidden_ref, w_ref, b_ref, out_ref):
    nh = 0.5 * lax.rsqrt(_deg_col(deg_ref[...]))
    hid = hidden_ref[...]
    cur_a = nh * (sa_ref[0, :N] + sa_ref[1, :N]) + 0.5 * hid[:, :DH]
    cur_b = nh * (sb_ref[0, :N] + sb_ref[1, :N]) + 0.5 * hid[:, DH:]
    cur = jnp.concatenate([cur_a, cur_b], axis=1)
    out = jnp.dot(cur, w_ref[...], preferred_element_type=jnp.float32)
    out_ref[...] = out + b_ref[...][None, :]


_prep_tc = pl.pallas_call(
    _prep_body,
    out_shape=(jax.ShapeDtypeStruct((N, D), jnp.float32),
               jax.ShapeDtypeStruct((N, DH), jnp.float32),
               jax.ShapeDtypeStruct((N, DH), jnp.float32)),
)

_comb_tc = pl.pallas_call(
    _comb_body,
    out_shape=(jax.ShapeDtypeStruct((N, DH), jnp.float32),
               jax.ShapeDtypeStruct((N, DH), jnp.float32)),
)

_final_tc = pl.pallas_call(
    _final_body,
    out_shape=jax.ShapeDtypeStruct((N, D), jnp.float32),
)


def kernel(x, edge_index, W_pre, b_pre, W_post, b_post):
    src = edge_index[0].astype(jnp.int32).reshape(NW, CH, K)
    dst = edge_index[1].astype(jnp.int32).reshape(NW, CH, K)
    zeros_nd = jnp.zeros((NB, SR, DH), jnp.float32)
    zeros_n16 = jnp.zeros((NB, SR, DW), jnp.float32)
    ones_k16 = jnp.ones((K, DW), jnp.float32)

    deg16 = _deg_sc(dst, ones_k16, zeros_n16)
    hidden, v0a, v0b = _prep_tc(x, W_pre, b_pre, deg16)
    va, vb = v0a, v0b
    for _ in range(STEPS - 1):
        sa, sb = _prop_sc(va, vb, src, dst, zeros_nd)
        va, vb = _comb_tc(sa, sb, deg16, v0a, v0b)
    sa, sb = _prop_sc(va, vb, src, dst, zeros_nd)
    return _final_tc(sa, sb, deg16, hidden, W_post, b_post)


# speedup vs baseline: 11.8212x; 1.5616x over previous
import functools

import jax
import jax.numpy as jnp
from jax import lax
from jax.experimental import pallas as pl
from jax.experimental.pallas import tpu as pltpu
from jax.experimental.pallas import tpu_sc as plsc

N = 10000
E = 320000
D = 128
DH = 64
STEPS = 4

NC = 2
NS = 16
NW = NC * NS
EPT = E // NW
K = 80
CH = EPT // K
NP = 10240
RPT = NP // NS
SR = 128
NB = NP // SR
BPT = RPT // SR
DW = 16

_mesh = plsc.VectorSubcoreMesh(core_axis_name="c", subcore_axis_name="s")


@functools.partial(
    pl.kernel,
    out_type=jax.ShapeDtypeStruct((NC, NP, DW), jnp.float32),
    mesh=_mesh,
    compiler_params=pltpu.CompilerParams(use_tc_tiling_on_sc=False),
    scratch_types=[
        pltpu.VMEM((CH, K), jnp.int32),
        pltpu.VMEM((K, DW), jnp.float32),
        pltpu.VMEM((SR, DW), jnp.float32),
        pltpu.VMEM_SHARED((NP, DW), jnp.float32),
    ],
)
def _deg_sc(dst_hbm, ones_hbm, zeros_hbm, out_hbm, dst_v, ones_v, stage_v, acc):
    cid = lax.axis_index("c")
    sid = lax.axis_index("s")
    wid = cid * NS + sid
    pltpu.sync_copy(dst_hbm.at[wid], dst_v)
    pltpu.sync_copy(ones_hbm, ones_v)
    for j in range(BPT):
        b = sid * BPT + j
        r0 = pl.multiple_of(b * SR, SR)
        pltpu.sync_copy(zeros_hbm.at[b], stage_v)
        pltpu.sync_copy(stage_v, acc.at[pl.ds(r0, SR)])
    plsc.subcore_barrier()

    def body(j, carry):
        pltpu.sync_copy(ones_v, acc.at[dst_v.at[j]], add=True)
        return carry

    lax.fori_loop(0, CH, body, 0)
    plsc.subcore_barrier()
    for j in range(BPT):
        b = sid * BPT + j
        r0 = pl.multiple_of(b * SR, SR)
        pltpu.sync_copy(acc.at[pl.ds(r0, SR)], stage_v)
        pltpu.sync_copy(stage_v, out_hbm.at[cid, pl.ds(r0, SR)])


@functools.partial(
    pl.kernel,
    out_type=(jax.ShapeDtypeStruct((NC, NP, DH), jnp.float32),
              jax.ShapeDtypeStruct((NC, NP, DH), jnp.float32)),
    mesh=_mesh,
    compiler_params=pltpu.CompilerParams(use_tc_tiling_on_sc=False),
    scratch_types=[
        pltpu.VMEM((CH, K), jnp.int32),
        pltpu.VMEM((CH, K), jnp.int32),
        pltpu.VMEM((K, DH), jnp.float32),
        pltpu.VMEM((K, DH), jnp.float32),
        pltpu.VMEM((SR, DH), jnp.float32),
        pltpu.VMEM_SHARED((NP, DH), jnp.float32),
        pltpu.SemaphoreType.DMA,
        pltpu.SemaphoreType.DMA,
    ],
)
def _prop_sc(va_hbm, vb_hbm, src_hbm, dst_hbm, zeros_hbm, outa_hbm, outb_hbm,
             src_v, dst_v, rows0_v, rows1_v, stage_v, acc, sem0, sem1):
    cid = lax.axis_index("c")
    sid = lax.axis_index("s")
    wid = cid * NS + sid
    pltpu.sync_copy(src_hbm.at[wid], src_v)
    pltpu.sync_copy(dst_hbm.at[wid], dst_v)
    for v_hbm, out_hbm in ((va_hbm, outa_hbm), (vb_hbm, outb_hbm)):
        for j in range(BPT):
            b = sid * BPT + j
            r0 = pl.multiple_of(b * SR, SR)
            pltpu.sync_copy(zeros_hbm.at[b], stage_v)
            pltpu.sync_copy(stage_v, acc.at[pl.ds(r0, SR)])
        plsc.subcore_barrier()

        pltpu.async_copy(v_hbm.at[src_v.at[0]], rows0_v, sem0)

        def body(i, carry):
            j = 2 * i
            pltpu.async_copy(v_hbm.at[src_v.at[j + 1]], rows1_v, sem1)
            pltpu.make_async_copy(v_hbm.at[src_v.at[j]], rows0_v, sem0).wait()
            pltpu.sync_copy(rows0_v, acc.at[dst_v.at[j]], add=True)
            pltpu.async_copy(v_hbm.at[src_v.at[j + 2]], rows0_v, sem0)
            pltpu.make_async_copy(v_hbm.at[src_v.at[j + 1]], rows1_v, sem1).wait()
            pltpu.sync_copy(rows1_v, acc.at[dst_v.at[j + 1]], add=True)
            return carry

        lax.fori_loop(0, (CH - 1) // 2, body, 0)
        pltpu.make_async_copy(v_hbm.at[src_v.at[CH - 1]], rows0_v, sem0).wait()
        pltpu.sync_copy(rows0_v, acc.at[dst_v.at[CH - 1]], add=True)
        plsc.subcore_barrier()
        for j in range(BPT):
            b = sid * BPT + j
            r0 = pl.multiple_of(b * SR, SR)
            pltpu.sync_copy(acc.at[pl.ds(r0, SR)], stage_v)
            pltpu.sync_copy(stage_v, out_hbm.at[cid, pl.ds(r0, SR)])


def _deg_col(deg16):
    d = (jnp.max(deg16[0, :N], axis=1, keepdims=True)
         + jnp.max(deg16[1, :N], axis=1, keepdims=True))
    return jnp.maximum(d, 1.0)


def _prep_body(x_ref, w_ref, b_ref, deg_ref, hidden_ref, v0a_ref, v0b_ref):
    h = jnp.dot(x_ref[...], w_ref[...], preferred_element_type=jnp.float32)
    h = h + b_ref[...][None, :]
    norm = lax.rsqrt(_deg_col(deg_ref[...]))
    hidden_ref[...] = h
    v0 = h * norm
    v0a_ref[...] = v0[:, :DH]
    v0b_ref[...] = v0[:, DH:]


def _comb_body(sa_ref, sb_ref, deg_ref, v0a_ref, v0b_ref, va_ref, vb_ref):
    inv = 0.5 / _deg_col(deg_ref[...])
    va_ref[...] = inv * (sa_ref[0, :N] + sa_ref[1, :N]) + 0.5 * v0a_ref[...]
    vb_ref[...] = inv * (sb_ref[0, :N] + sb_ref[1, :N]) + 0.5 * v0b_ref[...]


def _final_body(sa_ref, sb_ref, deg_ref, hidden_ref, w_ref, b_ref, out_ref):
    nh = 0.5 * lax.rsqrt(_deg_col(deg_ref[...]))
    hid = hidden_ref[...]
    cur_a = nh * (sa_ref[0, :N] + sa_ref[1, :N]) + 0.5 * hid[:, :DH]
    cur_b = nh * (sb_ref[0, :N] + sb_ref[1, :N]) + 0.5 * hid[:, DH:]
    cur = jnp.concatenate([cur_a, cur_b], axis=1)
    out = jnp.dot(cur, w_ref[...], preferred_element_type=jnp.float32)
    out_ref[...] = out + b_ref[...][None, :]


_prep_tc = pl.pallas_call(
    _prep_body,
    out_shape=(jax.ShapeDtypeStruct((N, D), jnp.float32),
               jax.ShapeDtypeStruct((N, DH), jnp.float32),
               jax.ShapeDtypeStruct((N, DH), jnp.float32)),
)

_comb_tc = pl.pallas_call(
    _comb_body,
    out_shape=(jax.ShapeDtypeStruct((N, DH), jnp.float32),
               jax.ShapeDtypeStruct((N, DH), jnp.float32)),
)

_final_tc = pl.pallas_call(
    _final_body,
    out_shape=jax.ShapeDtypeStruct((N, D), jnp.float32),
)


def kernel(x, edge_index, W_pre, b_pre, W_post, b_post):
    src = edge_index[0].astype(jnp.int32).reshape(NW, CH, K)
    dst = edge_index[1].astype(jnp.int32).reshape(NW, CH, K)
    zeros_nd = jnp.zeros((NB, SR, DH), jnp.float32)
    zeros_n16 = jnp.zeros((NB, SR, DW), jnp.float32)
    ones_k16 = jnp.ones((K, DW), jnp.float32)

    deg16 = _deg_sc(dst, ones_k16, zeros_n16)
    hidden, v0a, v0b = _prep_tc(x, W_pre, b_pre, deg16)
    va, vb = v0a, v0b
    for _ in range(STEPS - 1):
        sa, sb = _prop_sc(va, vb, src, dst, zeros_nd)
        va, vb = _comb_tc(sa, sb, deg16, v0a, v0b)
    sa, sb = _prop_sc(va, vb, src, dst, zeros_nd)
    return _final_tc(sa, sb, deg16, hidden, W_post, b_post)
